# Initial kernel scaffold; baseline (speedup 1.0000x reference)
import jax
import jax.numpy as jnp
from jax.experimental import pallas as pl


def kernel(x, edge_index, W1, b1, g1, be1, W2, b2, g2, be2):
    raise NotImplementedError("write your pallas kernel here")


# speedup vs baseline: 4.3532x; 4.3532x over previous
import functools

import jax
import jax.numpy as jnp
from jax import lax
from jax.experimental import pallas as pl
from jax.experimental.pallas import tpu as pltpu
from jax.experimental.pallas import tpu_sc as plsc

N_NODES = 10000
DIM = 256
HIDDEN = 512
N_EDGES = 160000
HALF = DIM // 2
N_SUBCORES = 16
CHUNK = 128
CHUNKS_PER_TILE = 80
EDGES_PER_TILE = CHUNK * CHUNKS_PER_TILE
EDGES_PAD = EDGES_PER_TILE * N_SUBCORES
ROWS_PAD = 10240
ROWS_PER_TILE = ROWS_PAD // N_SUBCORES


def _sc_body(x0h, x1h, srch, dsth, zh, out0, out1, acc, src_v, dst_v, rows_v,
             sem):
    c = lax.axis_index("c")
    s = lax.axis_index("s")
    base_r = s * ROWS_PER_TILE

    pltpu.sync_copy(zh, rows_v)
    for j in range(ROWS_PER_TILE // CHUNK):
        pltpu.sync_copy(rows_v, acc.at[pl.ds(base_r + j * CHUNK, CHUNK)])
    plsc.subcore_barrier()

    def edge_loop(xh):
        base_e = s * EDGES_PER_TILE

        def body(j, carry):
            off = base_e + j * CHUNK
            pltpu.sync_copy(srch.at[pl.ds(off, CHUNK)], src_v)
            pltpu.sync_copy(dsth.at[pl.ds(off, CHUNK)], dst_v)
            pltpu.async_copy(xh.at[src_v], rows_v, sem).wait()
            pltpu.sync_copy(rows_v, acc.at[dst_v], add=True)
            return carry

        lax.fori_loop(0, CHUNKS_PER_TILE, body, 0)

    @pl.when(c == 0)
    def _():
        edge_loop(x0h)

    @pl.when(c == 1)
    def _():
        edge_loop(x1h)

    plsc.subcore_barrier()

    def write_out(outh):
        for j in range(ROWS_PER_TILE // CHUNK):
            sl = pl.ds(base_r + j * CHUNK, CHUNK)
            pltpu.sync_copy(acc.at[sl], rows_v)
            pltpu.sync_copy(rows_v, outh.at[sl])

    @pl.when(c == 0)
    def _():
        write_out(out0)

    @pl.when(c == 1)
    def _():
        write_out(out1)


@functools.cache
def _sc_agg_fn():
    return pl.kernel(
        _sc_body,
        out_type=(
            jax.ShapeDtypeStruct((ROWS_PAD, HALF), jnp.float32),
            jax.ShapeDtypeStruct((ROWS_PAD, HALF), jnp.float32),
        ),
        mesh=plsc.VectorSubcoreMesh(core_axis_name="c", subcore_axis_name="s"),
        scratch_types=[
            pltpu.VMEM_SHARED((ROWS_PAD, HALF), jnp.float32),
            pltpu.VMEM((CHUNK,), jnp.int32),
            pltpu.VMEM((CHUNK,), jnp.int32),
            pltpu.VMEM((CHUNK, HALF), jnp.float32),
            pltpu.SemaphoreType.DMA,
        ],
    )


def _mlp_body(x_ref, a0_ref, a1_ref, w1_ref, b1_ref, g1_ref, be1_ref, w2_ref,
              b2_ref, g2_ref, be2_ref, o_ref, h_ref):
    x = x_ref[...]
    xa0 = x[:, :HALF] + a0_ref[0:N_NODES, :]
    xa1 = x[:, HALF:] + a1_ref[0:N_NODES, :]
    h = jnp.dot(xa0, w1_ref[0:HALF, :], preferred_element_type=jnp.float32)
    h = h + jnp.dot(xa1, w1_ref[HALF:DIM, :],
                    preferred_element_type=jnp.float32)
    h = h + b1_ref[...]
    h_ref[...] = h
    mean = jnp.mean(h_ref[...], axis=0, keepdims=True)
    var = jnp.mean((h_ref[...] - mean) ** 2, axis=0, keepdims=True)
    hn = (h_ref[...] - mean) * lax.rsqrt(var + 1e-5) * g1_ref[...] \
        + be1_ref[...]
    h_ref[...] = jnp.maximum(hn, 0.0)
    h2 = jnp.dot(h_ref[...], w2_ref[...], preferred_element_type=jnp.float32)
    h2 = h2 + b2_ref[...]
    mean2 = jnp.mean(h2, axis=0, keepdims=True)
    var2 = jnp.mean((h2 - mean2) ** 2, axis=0, keepdims=True)
    hn2 = (h2 - mean2) * lax.rsqrt(var2 + 1e-5) * g2_ref[...] + be2_ref[...]
    o_ref[...] = x + jnp.maximum(hn2, 0.0)


def _mlp(x, a0, a1, W1, b1, g1, be1, W2, b2, g2, be2, interpret=False):
    return pl.pallas_call(
        _mlp_body,
        out_shape=jax.ShapeDtypeStruct((N_NODES, DIM), jnp.float32),
        scratch_shapes=[pltpu.VMEM((N_NODES, HIDDEN), jnp.float32)],
        interpret=interpret,
    )(x, a0, a1, W1, b1.reshape(1, HIDDEN), g1.reshape(1, HIDDEN),
      be1.reshape(1, HIDDEN), W2, b2.reshape(1, DIM), g2.reshape(1, DIM),
      be2.reshape(1, DIM))


def kernel(x, edge_index, W1, b1, g1, be1, W2, b2, g2, be2):
    src = edge_index[0].astype(jnp.int32)
    dst = edge_index[1].astype(jnp.int32)
    pad = EDGES_PAD - N_EDGES
    pad_iota = jnp.arange(pad, dtype=jnp.int32)
    srcp = jnp.concatenate([src, pad_iota % N_NODES])
    dstp = jnp.concatenate([dst, N_NODES + pad_iota % (ROWS_PAD - N_NODES)])
    x0 = x[:, :HALF]
    x1 = x[:, HALF:]
    zeros_blk = jnp.zeros((CHUNK, HALF), jnp.float32)
    a0, a1 = _sc_agg_fn()(x0, x1, srcp, dstp, zeros_blk)
    return _mlp(x, a0, a1, W1, b1, g1, be1, W2, b2, g2, be2)


# speedup vs baseline: 7.8993x; 1.8146x over previous
import functools

import jax
import jax.numpy as jnp
from jax import lax
from jax.experimental import pallas as pl
from jax.experimental.pallas import tpu as pltpu
from jax.experimental.pallas import tpu_sc as plsc

N_NODES = 10000
DIM = 256
HIDDEN = 512
N_EDGES = 160000
HALF = DIM // 2
N_SUBCORES = 16
CHUNK = 128
CHUNKS_PER_TILE = 80
EDGES_PER_TILE = CHUNK * CHUNKS_PER_TILE
EDGES_PAD = EDGES_PER_TILE * N_SUBCORES
ROWS_PAD = 10240
ROWS_PER_TILE = ROWS_PAD // N_SUBCORES


NBUF = 2
NIDX = 4


def _sc_body(x0h, x1h, srch, dsth, zh, out0, out1, acc, src_v, dst_v, rows_v,
             gs0, gs1, ss0, ss1, is0, is1, is2, is3):
    c = lax.axis_index("c")
    s = lax.axis_index("s")
    base_r = s * ROWS_PER_TILE
    gsem = (gs0, gs1)
    ssem = (ss0, ss1)
    isem = (is0, is1, is2, is3)

    pltpu.sync_copy(zh, rows_v.at[0])
    for j in range(ROWS_PER_TILE // CHUNK):
        pltpu.sync_copy(rows_v.at[0], acc.at[pl.ds(base_r + j * CHUNK, CHUNK)])
    plsc.subcore_barrier()

    def edge_loop(xh):
        base_c = s * CHUNKS_PER_TILE

        def idx_descs(j, i):
            row = base_c + j
            return (pltpu.make_async_copy(srch.at[row], src_v.at[i], isem[i]),
                    pltpu.make_async_copy(dsth.at[row], dst_v.at[i], isem[i]))

        def fire_idx(j, i):
            row = base_c + j
            pltpu.async_copy(srch.at[row], src_v.at[i], isem[i])
            pltpu.async_copy(dsth.at[row], dst_v.at[i], isem[i])

        def wait_idx(j, i):
            a, b = idx_descs(j, i)
            a.wait()
            b.wait()

        def gather_desc(i, b):
            return pltpu.make_async_copy(xh.at[src_v.at[i]], rows_v.at[b],
                                         gsem[b])

        def scatter_desc(i, b):
            return pltpu.make_async_copy(rows_v.at[b], acc.at[dst_v.at[i]],
                                         ssem[b])

        def fire_gather(i, b):
            pltpu.async_copy(xh.at[src_v.at[i]], rows_v.at[b], gsem[b])

        def fire_scatter(i, b):
            pltpu.async_copy(rows_v.at[b], acc.at[dst_v.at[i]], ssem[b],
                             add=True)

        fire_idx(0, 0)
        fire_idx(1, 1)
        wait_idx(0, 0)
        fire_gather(0, 0)
        fire_idx(2, 2)
        wait_idx(1, 1)
        fire_gather(1, 1)
        fire_idx(3, 3)
        gather_desc(0, 0).wait()
        fire_scatter(0, 0)

        def body(k, carry):
            j0 = 2 + k * 4
            for b in range(4):
                j = j0 + b
                S = b % NBUF
                I = (2 + b) % NIDX
                Sp = (S + 1) % NBUF
                Ip = (I + 3) % NIDX
                scatter_desc((I + 2) % NIDX, S).wait()
                wait_idx(j, I)
                fire_gather(I, S)
                fire_idx(j + 2, (I + 2) % NIDX)
                gather_desc(Ip, Sp).wait()
                fire_scatter(Ip, Sp)
            return carry

        lax.fori_loop(0, (CHUNKS_PER_TILE - 4) // 4, body, 0)

        for j in (78, 79):
            S = j % NBUF
            I = j % NIDX
            Sp = (S + 1) % NBUF
            Ip = (I + 3) % NIDX
            scatter_desc((I + 2) % NIDX, S).wait()
            wait_idx(j, I)
            fire_gather(I, S)
            gather_desc(Ip, Sp).wait()
            fire_scatter(Ip, Sp)
        gather_desc(79 % NIDX, 79 % NBUF).wait()
        fire_scatter(79 % NIDX, 79 % NBUF)
        scatter_desc(78 % NIDX, 0).wait()
        scatter_desc(79 % NIDX, 1).wait()

    @pl.when(c == 0)
    def _():
        edge_loop(x0h)

    @pl.when(c == 1)
    def _():
        edge_loop(x1h)

    plsc.subcore_barrier()

    def write_out(outh):
        for j in range(ROWS_PER_TILE // CHUNK):
            sl = pl.ds(base_r + j * CHUNK, CHUNK)
            pltpu.sync_copy(acc.at[sl], rows_v.at[0])
            pltpu.sync_copy(rows_v.at[0], outh.at[sl])

    @pl.when(c == 0)
    def _():
        write_out(out0)

    @pl.when(c == 1)
    def _():
        write_out(out1)


@functools.cache
def _sc_agg_fn():
    return pl.kernel(
        _sc_body,
        out_type=(
            jax.ShapeDtypeStruct((ROWS_PAD, HALF), jnp.float32),
            jax.ShapeDtypeStruct((ROWS_PAD, HALF), jnp.float32),
        ),
        mesh=plsc.VectorSubcoreMesh(core_axis_name="c", subcore_axis_name="s"),
        scratch_types=[
            pltpu.VMEM_SHARED((ROWS_PAD, HALF), jnp.float32),
            pltpu.VMEM((NIDX, CHUNK), jnp.int32),
            pltpu.VMEM((NIDX, CHUNK), jnp.int32),
            pltpu.VMEM((NBUF, CHUNK, HALF), jnp.float32),
        ] + [pltpu.SemaphoreType.DMA] * (2 * NBUF + NIDX),
    )


def _mlp_body(x_ref, a0_ref, a1_ref, w1_ref, b1_ref, g1_ref, be1_ref, w2_ref,
              b2_ref, g2_ref, be2_ref, o_ref, h_ref):
    x = x_ref[...]
    xa0 = x[:, :HALF] + a0_ref[0:N_NODES, :]
    xa1 = x[:, HALF:] + a1_ref[0:N_NODES, :]
    h = jnp.dot(xa0, w1_ref[0:HALF, :], preferred_element_type=jnp.float32)
    h = h + jnp.dot(xa1, w1_ref[HALF:DIM, :],
                    preferred_element_type=jnp.float32)
    h = h + b1_ref[...]
    h_ref[...] = h
    mean = jnp.mean(h_ref[...], axis=0, keepdims=True)
    var = jnp.mean((h_ref[...] - mean) ** 2, axis=0, keepdims=True)
    hn = (h_ref[...] - mean) * lax.rsqrt(var + 1e-5) * g1_ref[...] \
        + be1_ref[...]
    h_ref[...] = jnp.maximum(hn, 0.0)
    h2 = jnp.dot(h_ref[...], w2_ref[...], preferred_element_type=jnp.float32)
    h2 = h2 + b2_ref[...]
    mean2 = jnp.mean(h2, axis=0, keepdims=True)
    var2 = jnp.mean((h2 - mean2) ** 2, axis=0, keepdims=True)
    hn2 = (h2 - mean2) * lax.rsqrt(var2 + 1e-5) * g2_ref[...] + be2_ref[...]
    o_ref[...] = x + jnp.maximum(hn2, 0.0)


def _mlp(x, a0, a1, W1, b1, g1, be1, W2, b2, g2, be2, interpret=False):
    return pl.pallas_call(
        _mlp_body,
        out_shape=jax.ShapeDtypeStruct((N_NODES, DIM), jnp.float32),
        scratch_shapes=[pltpu.VMEM((N_NODES, HIDDEN), jnp.float32)],
        interpret=interpret,
    )(x, a0, a1, W1, b1.reshape(1, HIDDEN), g1.reshape(1, HIDDEN),
      be1.reshape(1, HIDDEN), W2, b2.reshape(1, DIM), g2.reshape(1, DIM),
      be2.reshape(1, DIM))


def kernel(x, edge_index, W1, b1, g1, be1, W2, b2, g2, be2):
    src = edge_index[0].astype(jnp.int32)
    dst = edge_index[1].astype(jnp.int32)
    pad = EDGES_PAD - N_EDGES
    pad_iota = jnp.arange(pad, dtype=jnp.int32)
    srcp = jnp.concatenate([src, pad_iota % N_NODES]).reshape(-1, CHUNK)
    dstp = jnp.concatenate(
        [dst, N_NODES + pad_iota % (ROWS_PAD - N_NODES)]).reshape(-1, CHUNK)
    x0 = x[:, :HALF]
    x1 = x[:, HALF:]
    zeros_blk = jnp.zeros((CHUNK, HALF), jnp.float32)
    a0, a1 = _sc_agg_fn()(x0, x1, srcp, dstp, zeros_blk)
    return _mlp(x, a0, a1, W1, b1, g1, be1, W2, b2, g2, be2)


# speedup vs baseline: 8.0889x; 1.0240x over previous
import functools

import jax
import jax.numpy as jnp
from jax import lax
from jax.experimental import pallas as pl
from jax.experimental.pallas import tpu as pltpu
from jax.experimental.pallas import tpu_sc as plsc

N_NODES = 10000
DIM = 256
HIDDEN = 512
N_EDGES = 160000
HALF = DIM // 2
N_SUBCORES = 16
CHUNK = 128
CHUNKS_PER_TILE = 80
EDGES_PER_TILE = CHUNK * CHUNKS_PER_TILE
EDGES_PAD = EDGES_PER_TILE * N_SUBCORES
ROWS_PAD = 10240
ROWS_PER_TILE = ROWS_PAD // N_SUBCORES


NBUF = 2
NIDX = 4


def _sc_body(xh, srch, dsth, zh, outh, acc, src_v, dst_v, rows_v,
             gs0, gs1, ss0, ss1, is0, is1, is2, is3):
    c = lax.axis_index("c")
    s = lax.axis_index("s")
    base_r = s * ROWS_PER_TILE
    base_c = s * CHUNKS_PER_TILE
    gsem = (gs0, gs1)
    ssem = (ss0, ss1)
    isem = (is0, is1, is2, is3)

    def idx_descs(j, i):
        row = base_c + j
        return (pltpu.make_async_copy(srch.at[row], src_v.at[i], isem[i]),
                pltpu.make_async_copy(dsth.at[row], dst_v.at[i], isem[i]))

    def fire_idx(j, i):
        row = base_c + j
        pltpu.async_copy(srch.at[row], src_v.at[i], isem[i])
        pltpu.async_copy(dsth.at[row], dst_v.at[i], isem[i])

    def wait_idx(j, i):
        a, b = idx_descs(j, i)
        a.wait()
        b.wait()

    def gather_desc(col, i, b):
        return pltpu.make_async_copy(
            xh.at[src_v.at[i], pl.ds(col, HALF)], rows_v.at[b], gsem[b])

    def scatter_desc(i, b):
        return pltpu.make_async_copy(rows_v.at[b], acc.at[dst_v.at[i]],
                                     ssem[b])

    def fire_gather(col, i, b):
        pltpu.async_copy(xh.at[src_v.at[i], pl.ds(col, HALF)], rows_v.at[b],
                         gsem[b])

    def fire_scatter(i, b):
        pltpu.async_copy(rows_v.at[b], acc.at[dst_v.at[i]], ssem[b], add=True)

    pltpu.sync_copy(zh, rows_v.at[0])
    for j in range(ROWS_PER_TILE // CHUNK):
        pltpu.sync_copy(rows_v.at[0], acc.at[pl.ds(base_r + j * CHUNK, CHUNK)])

    def prologue(col):
        fire_idx(0, 0)
        fire_idx(1, 1)
        wait_idx(0, 0)
        fire_gather(col, 0, 0)
        fire_idx(2, 2)
        wait_idx(1, 1)
        fire_gather(col, 1, 1)
        fire_idx(3, 3)

    @pl.when(c == 0)
    def _():
        prologue(0)

    @pl.when(c == 1)
    def _():
        prologue(HALF)

    plsc.subcore_barrier()

    def main(col):
        gather_desc(col, 0, 0).wait()
        fire_scatter(0, 0)

        def body(k, carry):
            j0 = 2 + k * 4
            for b in range(4):
                j = j0 + b
                S = b % NBUF
                I = (2 + b) % NIDX
                Sp = (S + 1) % NBUF
                Ip = (I + 3) % NIDX
                scatter_desc((I + 2) % NIDX, S).wait()
                wait_idx(j, I)
                fire_gather(col, I, S)
                fire_idx(j + 2, (I + 2) % NIDX)
                gather_desc(col, Ip, Sp).wait()
                fire_scatter(Ip, Sp)
            return carry

        lax.fori_loop(0, (CHUNKS_PER_TILE - 4) // 4, body, 0)

        for j in (78, 79):
            S = j % NBUF
            I = j % NIDX
            Sp = (S + 1) % NBUF
            Ip = (I + 3) % NIDX
            scatter_desc((I + 2) % NIDX, S).wait()
            wait_idx(j, I)
            fire_gather(col, I, S)
            gather_desc(col, Ip, Sp).wait()
            fire_scatter(Ip, Sp)
        gather_desc(col, 79 % NIDX, 79 % NBUF).wait()
        fire_scatter(79 % NIDX, 79 % NBUF)
        scatter_desc(78 % NIDX, 0).wait()
        scatter_desc(79 % NIDX, 1).wait()

    @pl.when(c == 0)
    def _():
        main(0)

    @pl.when(c == 1)
    def _():
        main(HALF)

    plsc.subcore_barrier()

    def write_out(col):
        for j in range(ROWS_PER_TILE // CHUNK):
            sl = pl.ds(base_r + j * CHUNK, CHUNK)
            pltpu.sync_copy(acc.at[sl], rows_v.at[0])
            pltpu.sync_copy(rows_v.at[0], outh.at[sl, pl.ds(col, HALF)])

    @pl.when(c == 0)
    def _():
        write_out(0)

    @pl.when(c == 1)
    def _():
        write_out(HALF)


@functools.cache
def _sc_agg_fn():
    return pl.kernel(
        _sc_body,
        out_type=jax.ShapeDtypeStruct((ROWS_PAD, DIM), jnp.float32),
        mesh=plsc.VectorSubcoreMesh(core_axis_name="c", subcore_axis_name="s"),
        scratch_types=[
            pltpu.VMEM_SHARED((ROWS_PAD, HALF), jnp.float32),
            pltpu.VMEM((NIDX, CHUNK), jnp.int32),
            pltpu.VMEM((NIDX, CHUNK), jnp.int32),
            pltpu.VMEM((NBUF, CHUNK, HALF), jnp.float32),
        ] + [pltpu.SemaphoreType.DMA] * (2 * NBUF + NIDX),
    )


def _mlp_body(x_ref, a_ref, w1_ref, b1_ref, g1_ref, be1_ref, w2_ref,
              b2_ref, g2_ref, be2_ref, o_ref, h_ref):
    x = x_ref[...]
    xa = x + a_ref[0:N_NODES, :]
    h = jnp.dot(xa, w1_ref[...], preferred_element_type=jnp.float32)
    h = h + b1_ref[...]
    h_ref[...] = h
    mean = jnp.mean(h_ref[...], axis=0, keepdims=True)
    var = jnp.mean((h_ref[...] - mean) ** 2, axis=0, keepdims=True)
    hn = (h_ref[...] - mean) * lax.rsqrt(var + 1e-5) * g1_ref[...] \
        + be1_ref[...]
    h_ref[...] = jnp.maximum(hn, 0.0)
    h2 = jnp.dot(h_ref[...], w2_ref[...], preferred_element_type=jnp.float32)
    h2 = h2 + b2_ref[...]
    mean2 = jnp.mean(h2, axis=0, keepdims=True)
    var2 = jnp.mean((h2 - mean2) ** 2, axis=0, keepdims=True)
    hn2 = (h2 - mean2) * lax.rsqrt(var2 + 1e-5) * g2_ref[...] + be2_ref[...]
    o_ref[...] = x + jnp.maximum(hn2, 0.0)


def _mlp(x, a, W1, b1, g1, be1, W2, b2, g2, be2, interpret=False):
    return pl.pallas_call(
        _mlp_body,
        out_shape=jax.ShapeDtypeStruct((N_NODES, DIM), jnp.float32),
        scratch_shapes=[pltpu.VMEM((N_NODES, HIDDEN), jnp.float32)],
        interpret=interpret,
    )(x, a, W1, b1.reshape(1, HIDDEN), g1.reshape(1, HIDDEN),
      be1.reshape(1, HIDDEN), W2, b2.reshape(1, DIM), g2.reshape(1, DIM),
      be2.reshape(1, DIM))


def kernel(x, edge_index, W1, b1, g1, be1, W2, b2, g2, be2):
    src = edge_index[0].astype(jnp.int32)
    dst = edge_index[1].astype(jnp.int32)
    pad = EDGES_PAD - N_EDGES
    pad_iota = jnp.arange(pad, dtype=jnp.int32)
    srcp = jnp.concatenate([src, pad_iota % N_NODES]).reshape(-1, CHUNK)
    dstp = jnp.concatenate(
        [dst, N_NODES + pad_iota % (ROWS_PAD - N_NODES)]).reshape(-1, CHUNK)
    zeros_blk = jnp.zeros((CHUNK, HALF), jnp.float32)
    a = _sc_agg_fn()(x, srcp, dstp, zeros_blk)
    return _mlp(x, a, W1, b1, g1, be1, W2, b2, g2, be2)


# speedup vs baseline: 8.4715x; 1.0473x over previous
import functools

import jax
import jax.numpy as jnp
from jax import lax
from jax.experimental import pallas as pl
from jax.experimental.pallas import tpu as pltpu
from jax.experimental.pallas import tpu_sc as plsc

N_NODES = 10000
DIM = 256
HIDDEN = 512
N_EDGES = 160000
HALF = DIM // 2
N_SUBCORES = 16
CHUNK = 128
CHUNKS_PER_TILE = 80
EDGES_PER_TILE = CHUNK * CHUNKS_PER_TILE
EDGES_PAD = EDGES_PER_TILE * N_SUBCORES
ROWS_PAD = 10240
ROWS_PER_TILE = ROWS_PAD // N_SUBCORES


NBUF = 2
NIDX = 4


def _sc_body(xh, srch, dsth, zh, outh, acc, src_v, dst_v, rows_v,
             gs0, gs1, ss0, ss1, is0, is1, is2, is3):
    c = lax.axis_index("c")
    s = lax.axis_index("s")
    base_r = s * ROWS_PER_TILE
    base_c = s * CHUNKS_PER_TILE
    gsem = (gs0, gs1)
    ssem = (ss0, ss1)
    isem = (is0, is1, is2, is3)

    def idx_descs(j, i):
        row = base_c + j
        return (pltpu.make_async_copy(srch.at[row], src_v.at[i], isem[i]),
                pltpu.make_async_copy(dsth.at[row], dst_v.at[i], isem[i]))

    def fire_idx(j, i):
        row = base_c + j
        pltpu.async_copy(srch.at[row], src_v.at[i], isem[i])
        pltpu.async_copy(dsth.at[row], dst_v.at[i], isem[i])

    def wait_idx(j, i):
        a, b = idx_descs(j, i)
        a.wait()
        b.wait()

    def gather_desc(col, i, b):
        return pltpu.make_async_copy(
            xh.at[src_v.at[i], pl.ds(col, HALF)], rows_v.at[b], gsem[b])

    def scatter_desc(i, b):
        return pltpu.make_async_copy(rows_v.at[b], acc.at[dst_v.at[i]],
                                     ssem[b])

    def fire_gather(col, i, b):
        pltpu.async_copy(xh.at[src_v.at[i], pl.ds(col, HALF)], rows_v.at[b],
                         gsem[b])

    def fire_scatter(i, b):
        pltpu.async_copy(rows_v.at[b], acc.at[dst_v.at[i]], ssem[b], add=True)

    fire_idx(0, 0)
    fire_idx(1, 1)
    pltpu.sync_copy(zh, rows_v.at[0])
    for j in range(ROWS_PER_TILE // CHUNK):
        pltpu.sync_copy(rows_v.at[0], acc.at[pl.ds(base_r + j * CHUNK, CHUNK)])

    def prologue(col):
        wait_idx(0, 0)
        fire_gather(col, 0, 0)
        fire_idx(2, 2)
        wait_idx(1, 1)
        fire_gather(col, 1, 1)
        fire_idx(3, 3)

    @pl.when(c == 0)
    def _():
        prologue(0)

    @pl.when(c == 1)
    def _():
        prologue(HALF)

    plsc.subcore_barrier()

    def main(col):
        gather_desc(col, 0, 0).wait()
        fire_scatter(0, 0)

        def body(k, carry):
            j0 = 2 + k * 4
            for b in range(4):
                j = j0 + b
                S = b % NBUF
                I = (2 + b) % NIDX
                Sp = (S + 1) % NBUF
                Ip = (I + 3) % NIDX
                scatter_desc((I + 2) % NIDX, S).wait()
                wait_idx(j, I)
                fire_gather(col, I, S)
                fire_idx(j + 2, (I + 2) % NIDX)
                gather_desc(col, Ip, Sp).wait()
                fire_scatter(Ip, Sp)
            return carry

        lax.fori_loop(0, (CHUNKS_PER_TILE - 4) // 4, body, 0)

        for j in (78, 79):
            S = j % NBUF
            I = j % NIDX
            Sp = (S + 1) % NBUF
            Ip = (I + 3) % NIDX
            scatter_desc((I + 2) % NIDX, S).wait()
            wait_idx(j, I)
            fire_gather(col, I, S)
            gather_desc(col, Ip, Sp).wait()
            fire_scatter(Ip, Sp)
        gather_desc(col, 79 % NIDX, 79 % NBUF).wait()
        fire_scatter(79 % NIDX, 79 % NBUF)
        scatter_desc(78 % NIDX, 0).wait()
        scatter_desc(79 % NIDX, 1).wait()

    @pl.when(c == 0)
    def _():
        main(0)

    @pl.when(c == 1)
    def _():
        main(HALF)

    plsc.subcore_barrier()

    def write_out(col):
        nblk = ROWS_PER_TILE // CHUNK

        def load(j, b):
            sl = pl.ds(base_r + j * CHUNK, CHUNK)
            return pltpu.make_async_copy(acc.at[sl], rows_v.at[b], gsem[b])

        def store(j, b):
            sl = pl.ds(base_r + j * CHUNK, CHUNK)
            return pltpu.make_async_copy(
                rows_v.at[b], outh.at[sl, pl.ds(col, HALF)], ssem[b])

        pltpu.async_copy(acc.at[pl.ds(base_r, CHUNK)], rows_v.at[0], gsem[0])
        for j in range(nblk):
            b = j % NBUF
            bn = (j + 1) % NBUF
            load(j, b).wait()
            pltpu.async_copy(
                rows_v.at[b],
                outh.at[pl.ds(base_r + j * CHUNK, CHUNK), pl.ds(col, HALF)],
                ssem[b])
            if j + 1 < nblk:
                if j + 1 >= NBUF:
                    store(j + 1 - NBUF, bn).wait()
                pltpu.async_copy(acc.at[pl.ds(base_r + (j + 1) * CHUNK, CHUNK)],
                                 rows_v.at[bn], gsem[bn])
        store(nblk - 2, (nblk - 2) % NBUF).wait()
        store(nblk - 1, (nblk - 1) % NBUF).wait()

    @pl.when(c == 0)
    def _():
        write_out(0)

    @pl.when(c == 1)
    def _():
        write_out(HALF)


@functools.cache
def _sc_agg_fn():
    return pl.kernel(
        _sc_body,
        out_type=jax.ShapeDtypeStruct((ROWS_PAD, DIM), jnp.float32),
        mesh=plsc.VectorSubcoreMesh(core_axis_name="c", subcore_axis_name="s"),
        scratch_types=[
            pltpu.VMEM_SHARED((ROWS_PAD, HALF), jnp.float32),
            pltpu.VMEM((NIDX, CHUNK), jnp.int32),
            pltpu.VMEM((NIDX, CHUNK), jnp.int32),
            pltpu.VMEM((NBUF, CHUNK, HALF), jnp.float32),
        ] + [pltpu.SemaphoreType.DMA] * (2 * NBUF + NIDX),
    )


def _mlp_body(x_ref, a_ref, w1_ref, b1_ref, g1_ref, be1_ref, w2_ref,
              b2_ref, g2_ref, be2_ref, o_ref, h_ref, hb_ref):
    x = x_ref[...]
    xa = x + a_ref[0:N_NODES, :]
    h = jnp.dot(xa.astype(jnp.bfloat16), w1_ref[...],
                preferred_element_type=jnp.float32)
    h = h + b1_ref[...]
    h_ref[...] = h
    h = h_ref[...]
    n = jnp.float32(1.0 / N_NODES)
    mean = jnp.sum(h, axis=0, keepdims=True) * n
    sq = jnp.sum(h * h, axis=0, keepdims=True) * n
    var = sq - mean * mean
    scale = lax.rsqrt(var + 1e-5) * g1_ref[...]
    hn = (h - mean) * scale + be1_ref[...]
    hb_ref[...] = jnp.maximum(hn, 0.0).astype(jnp.bfloat16)
    h2 = jnp.dot(hb_ref[...], w2_ref[...], preferred_element_type=jnp.float32)
    h2 = h2 + b2_ref[...]
    mean2 = jnp.sum(h2, axis=0, keepdims=True) * n
    sq2 = jnp.sum(h2 * h2, axis=0, keepdims=True) * n
    var2 = sq2 - mean2 * mean2
    scale2 = lax.rsqrt(var2 + 1e-5) * g2_ref[...]
    hn2 = (h2 - mean2) * scale2 + be2_ref[...]
    o_ref[...] = x + jnp.maximum(hn2, 0.0)


def _mlp(x, a, W1, b1, g1, be1, W2, b2, g2, be2, interpret=False):
    return pl.pallas_call(
        _mlp_body,
        out_shape=jax.ShapeDtypeStruct((N_NODES, DIM), jnp.float32),
        scratch_shapes=[pltpu.VMEM((N_NODES, HIDDEN), jnp.float32),
                        pltpu.VMEM((N_NODES, HIDDEN), jnp.bfloat16)],
        interpret=interpret,
    )(x, a, W1.astype(jnp.bfloat16), b1.reshape(1, HIDDEN),
      g1.reshape(1, HIDDEN), be1.reshape(1, HIDDEN),
      W2.astype(jnp.bfloat16), b2.reshape(1, DIM), g2.reshape(1, DIM),
      be2.reshape(1, DIM))


def kernel(x, edge_index, W1, b1, g1, be1, W2, b2, g2, be2):
    src = edge_index[0].astype(jnp.int32)
    dst = edge_index[1].astype(jnp.int32)
    pad = EDGES_PAD - N_EDGES
    pad_iota = jnp.arange(pad, dtype=jnp.int32)
    srcp = jnp.concatenate([src, pad_iota % N_NODES]).reshape(-1, CHUNK)
    dstp = jnp.concatenate(
        [dst, N_NODES + pad_iota % (ROWS_PAD - N_NODES)]).reshape(-1, CHUNK)
    zeros_blk = jnp.zeros((CHUNK, HALF), jnp.float32)
    a = _sc_agg_fn()(x, srcp, dstp, zeros_blk)
    return _mlp(x, a, W1, b1, g1, be1, W2, b2, g2, be2)
